# baseline (speedup 1.0000x reference)
import functools

import jax
import jax.numpy as jnp
from jax import lax
from jax.experimental import pallas as pl
from jax.experimental.pallas import tpu as pltpu
from jax.experimental.pallas import tpu_sc as plsc

N_ROWS = 1000000
EMBED_DIM = 16
BATCH = 16384
HIST = 50

NC = 2
NS = 16
NW = NC * NS
TJ = BATCH // 128
TJ_PER_W = TJ // NW
BW = 128 * TJ_PER_W

CW = 2048
N_CHUNK_G = -(-N_ROWS // CW)
N_IT = -(-N_CHUNK_G // NW)


def _fmt_body(wt_hbm, wlin_hbm, in_a, in_b, out_a, out_b, sem_i, sem_w):
    wid = lax.axis_index("s") * NC + lax.axis_index("c")
    iota16 = lax.iota(jnp.int32, EMBED_DIM)

    def start(i, ibuf):
        c = jnp.minimum(wid + NW * i, N_CHUNK_G - 1)
        i0 = jnp.minimum(c * CW, N_ROWS - CW)
        for d in range(EMBED_DIM):
            pltpu.async_copy(wt_hbm.at[d, pl.ds(i0, CW)], ibuf.at[d], sem_i)
        return i0

    def drain_in(ibuf):
        for d in range(EMBED_DIM):
            pltpu.make_async_copy(
                wt_hbm.at[d, pl.ds(0, CW)], ibuf.at[d], sem_i
            ).wait()

    def transpose(ibuf, obuf):
        for d in range(EMBED_DIM):
            d_splat = iota16 * 0 + d

            def per_g(g, carry):
                vals = ibuf[d, pl.ds(g * 16, 16)]
                plsc.store_scatter(obuf, [g * 16 + iota16, d_splat], vals)
                return carry

            lax.fori_loop(0, CW // 16, per_g, 0, unroll=4)

    def write_out(i0, obuf):
        pltpu.async_copy(obuf, wlin_hbm.at[pl.ds(i0, CW)], sem_w)

    def drain_out(obuf):
        pltpu.make_async_copy(obuf, wlin_hbm.at[pl.ds(0, CW)], sem_w).wait()

    i0a = start(0, in_a)

    def step(k, carry):
        ia = carry
        ib = start(2 * k + 1, in_b)
        drain_in(in_a)

        @pl.when(k > 0)
        def _():
            drain_out(out_a)

        transpose(in_a, out_a)
        write_out(ia, out_a)

        ia_next = start(2 * k + 2, in_a)
        drain_in(in_b)

        @pl.when(k > 0)
        def _():
            drain_out(out_b)

        transpose(in_b, out_b)
        write_out(ib, out_b)
        return ia_next

    lax.fori_loop(0, N_IT // 2, step, i0a)

    drain_in(in_a)
    drain_out(out_a)
    drain_out(out_b)


def _body(
    xT_hbm, w_hbm, out_hbm,
    idx_a, idx_b, rows_a, rows_b, tbuf_a, tbuf_b,
    sem_g, sem_o,
):
    wid = lax.axis_index("s") * NC + lax.axis_index("c")
    base = wid * BW

    d_iota = lax.iota(jnp.int32, EMBED_DIM)
    ti_idx = d_iota // 8
    r_idx = d_iota % 8
    zeros = jnp.zeros((EMBED_DIM,), jnp.int32)

    def stage(h, idx_v, rows_v):
        pltpu.sync_copy(xT_hbm.at[h, pl.ds(base, BW)], idx_v)
        return pltpu.async_copy(w_hbm.at[idx_v], rows_v, sem_g)

    def transpose(rows_v, tbuf):
        for j in range(TJ_PER_W):
            j_splat = zeros + j

            def per_c(c, carry):
                vals = rows_v[j * 128 + c]
                plsc.store_scatter(
                    tbuf, [ti_idx, j_splat, r_idx, zeros + c], vals
                )
                return carry

            lax.fori_loop(0, 128, per_c, 0, unroll=8)

    def write_out(h, tbuf):
        pltpu.async_copy(
            tbuf.at[0], out_hbm.at[h, 0, pl.ds(wid * TJ_PER_W, TJ_PER_W)], sem_o
        )
        pltpu.async_copy(
            tbuf.at[1], out_hbm.at[h, 1, pl.ds(wid * TJ_PER_W, TJ_PER_W)], sem_o
        )

    def drain_writes(h, tbuf):
        pltpu.make_async_copy(
            tbuf.at[0], out_hbm.at[h, 0, pl.ds(wid * TJ_PER_W, TJ_PER_W)], sem_o
        ).wait()
        pltpu.make_async_copy(
            tbuf.at[1], out_hbm.at[h, 1, pl.ds(wid * TJ_PER_W, TJ_PER_W)], sem_o
        ).wait()

    stage(0, idx_a, rows_a)

    def step(k, carry):
        h0 = 2 * k
        h1 = 2 * k + 1
        gb = stage(h1, idx_b, rows_b)
        gb.wait()

        @pl.when(k > 0)
        def _():
            drain_writes(h0, tbuf_a)

        transpose(rows_a, tbuf_a)
        write_out(h0, tbuf_a)

        ga = stage(jnp.minimum(h1 + 1, HIST - 1), idx_a, rows_a)
        ga.wait()

        @pl.when(k > 0)
        def _():
            drain_writes(h1, tbuf_b)

        transpose(rows_b, tbuf_b)
        write_out(h1, tbuf_b)
        return carry

    lax.fori_loop(0, HIST // 2, step, 0)

    pltpu.make_async_copy(w_hbm.at[idx_a], rows_a, sem_g).wait()
    drain_writes(HIST - 2, tbuf_a)
    drain_writes(HIST - 1, tbuf_b)


_MESH = dict(core_axis_name="c", subcore_axis_name="s")
_PARAMS = dict(use_tc_tiling_on_sc=False, needs_layout_passes=False)


@jax.jit
def _run(xT, Wt):
    fmt = pl.kernel(
        _fmt_body,
        out_type=jax.ShapeDtypeStruct((N_ROWS, EMBED_DIM), jnp.float32),
        mesh=plsc.VectorSubcoreMesh(**_MESH),
        scratch_types=[
            pltpu.VMEM((EMBED_DIM, CW), jnp.float32),
            pltpu.VMEM((EMBED_DIM, CW), jnp.float32),
            pltpu.VMEM((CW, EMBED_DIM), jnp.float32),
            pltpu.VMEM((CW, EMBED_DIM), jnp.float32),
            pltpu.SemaphoreType.DMA,
            pltpu.SemaphoreType.DMA,
        ],
        compiler_params=pltpu.CompilerParams(**_PARAMS),
    )
    wlin = fmt(Wt)

    look = pl.kernel(
        _body,
        out_type=jax.ShapeDtypeStruct((HIST, 2, TJ, 8, 128), jnp.float32),
        mesh=plsc.VectorSubcoreMesh(**_MESH),
        scratch_types=[
            pltpu.VMEM((BW,), jnp.int32),
            pltpu.VMEM((BW,), jnp.int32),
            pltpu.VMEM((BW, EMBED_DIM), jnp.float32),
            pltpu.VMEM((BW, EMBED_DIM), jnp.float32),
            pltpu.VMEM((2, TJ_PER_W, 8, 128), jnp.float32),
            pltpu.VMEM((2, TJ_PER_W, 8, 128), jnp.float32),
            pltpu.SemaphoreType.DMA,
            pltpu.SemaphoreType.DMA,
        ],
        compiler_params=pltpu.CompilerParams(**_PARAMS),
    )
    return look(xT, wlin)


def kernel(x, W):
    out5 = _run(x.T, W.T)
    return out5.transpose(2, 4, 0, 1, 3).reshape(BATCH, HIST, EMBED_DIM)


# speedup vs baseline: 2.4462x; 2.4462x over previous
import functools

import jax
import jax.numpy as jnp
from jax import lax
from jax.experimental import pallas as pl
from jax.experimental.pallas import tpu as pltpu
from jax.experimental.pallas import tpu_sc as plsc

N_ROWS = 1000000
EMBED_DIM = 16
BATCH = 16384
HIST = 50

NC = 2
NS = 16
NW = NC * NS
TJ = BATCH // 128
TJ_PER_W = TJ // NW
BW = 128 * TJ_PER_W


def _body(
    xT_hbm, w_hbm, out_hbm,
    idx_all, rows_a, rows_b, tbuf_a, tbuf_b,
    sem_g, sem_o,
):
    wid = lax.axis_index("s") * NC + lax.axis_index("c")
    base = wid * BW

    d_iota = lax.iota(jnp.int32, EMBED_DIM)
    ti_idx = d_iota // 8
    r_idx = d_iota % 8
    zeros = jnp.zeros((EMBED_DIM,), jnp.int32)

    pltpu.sync_copy(xT_hbm.at[:, pl.ds(base, BW)], idx_all)

    def gather(h, rows_v):
        return pltpu.async_copy(w_hbm.at[idx_all.at[h]], rows_v, sem_g)

    def transpose(rows_v, tbuf):
        for j in range(TJ_PER_W):
            j_splat = zeros + j

            def per_c(c, carry):
                vals = rows_v[j * 128 + c]
                plsc.store_scatter(
                    tbuf, [ti_idx, j_splat, r_idx, zeros + c], vals
                )
                return carry

            lax.fori_loop(0, 128, per_c, 0, unroll=8)

    def write_out(h, tbuf):
        pltpu.async_copy(
            tbuf, out_hbm.at[h, :, pl.ds(wid * TJ_PER_W, TJ_PER_W)], sem_o
        )

    def drain_write(h, tbuf):
        pltpu.make_async_copy(
            tbuf, out_hbm.at[h, :, pl.ds(wid * TJ_PER_W, TJ_PER_W)], sem_o
        ).wait()

    gather(0, rows_a)

    def step(k, carry):
        h0 = 2 * k
        h1 = 2 * k + 1
        gather(h1, rows_b).wait()

        @pl.when(k > 0)
        def _():
            drain_write(h0, tbuf_a)

        transpose(rows_a, tbuf_a)
        write_out(h0, tbuf_a)

        gather(jnp.minimum(h1 + 1, HIST - 1), rows_a).wait()

        @pl.when(k > 0)
        def _():
            drain_write(h1, tbuf_b)

        transpose(rows_b, tbuf_b)
        write_out(h1, tbuf_b)
        return carry

    lax.fori_loop(0, HIST // 2, step, 0)

    pltpu.make_async_copy(w_hbm.at[idx_all.at[0]], rows_a, sem_g).wait()
    drain_write(HIST - 2, tbuf_a)
    drain_write(HIST - 1, tbuf_b)


@jax.jit
def _lookup(xT, W):
    k = pl.kernel(
        _body,
        out_type=jax.ShapeDtypeStruct((HIST, 2, TJ, 8, 128), jnp.float32),
        mesh=plsc.VectorSubcoreMesh(core_axis_name="c", subcore_axis_name="s"),
        scratch_types=[
            pltpu.VMEM((HIST, BW), jnp.int32),
            pltpu.VMEM((BW, EMBED_DIM), jnp.float32),
            pltpu.VMEM((BW, EMBED_DIM), jnp.float32),
            pltpu.VMEM((2, TJ_PER_W, 8, 128), jnp.float32),
            pltpu.VMEM((2, TJ_PER_W, 8, 128), jnp.float32),
            pltpu.SemaphoreType.DMA,
            pltpu.SemaphoreType.DMA,
        ],
        compiler_params=pltpu.CompilerParams(
            use_tc_tiling_on_sc=False, needs_layout_passes=False
        ),
    )
    return k(xT, W)


def kernel(x, W):
    out5 = _lookup(x.T, W)
    return out5.transpose(2, 4, 0, 1, 3).reshape(BATCH, HIST, EMBED_DIM)
